# baseline (speedup 1.0000x reference)
import math

import jax
import jax.numpy as jnp
from jax.experimental import pallas as pl
from jax.experimental.pallas import tpu as pltpu

_K_BUDGET = 128.0
_NUM_STAGES = 2
_R = 2
_GAMMA = 1.0
_EPS = 1e-6
_TILE_I = 512


def _dotg(a, b, contract):
    return jax.lax.dot_general(
        a, b, (contract, ((), ())), preferred_element_type=jnp.float32
    )


def _router_kernel(x_ref, wproj_ref, bproj_ref, wq_ref, wk_ref, bpos_ref,
                   lt_ref, m0_ref, wm_ref, bm_ref, out_ref, dist_scr):
    f32 = jnp.float32
    Tc = x_ref.shape[1]
    scale_a = math.sqrt(wq_ref.shape[0])

    xb = x_ref[0]
    u = _dotg(xb, wproj_ref[...], ((1,), (1,))) + bproj_ref[...]

    usq = u * u
    sq_col = jnp.sum(usq, axis=1, keepdims=True)
    ones_row = jnp.ones((1, u.shape[1]), f32)
    sq_row = _dotg(ones_row, usq, ((1,), (1,)))

    temp = jnp.clip(jnp.exp(lt_ref[0, 0]), 0.1, 10.0)
    lane_ids = jax.lax.broadcasted_iota(jnp.int32, (1, Tc), 1)
    positions = lane_ids.astype(f32)

    m = m0_ref[...]
    prev = jnp.zeros((1, Tc), f32)
    yl = prev
    for _stage in range(_NUM_STAGES):
        q = _dotg(m, wq_ref[...], ((1,), (1,)))
        k = _dotg(u, wk_ref[...], ((1,), (1,)))
        scores = _dotg(q, k, ((1,), (1,))) / scale_a
        scores = scores + bpos_ref[...] - _GAMMA * prev
        yl = jax.nn.sigmoid(scores / temp)
        budget = jnp.maximum(jnp.sum(yl), 1e-6)
        yl = yl * jnp.minimum(_K_BUDGET / budget, 1.0)
        for d in range(1, _R + 1):
            shift = jnp.concatenate([yl[:, d:], yl[:, :d]], axis=1)
            yl = yl * jnp.minimum(2.0 / (1.0 + yl + shift), 1.0)
        yl = jnp.where(lane_ids == 0, 0.0, yl)

        ssum = jnp.sum(yl, axis=1, keepdims=True)
        coverage = ssum / Tc
        ysum = jnp.maximum(ssum, _EPS)
        ynorm = yl / ysum
        entropy = -jnp.sum(ynorm * jnp.log(jnp.maximum(ynorm, _EPS)),
                           axis=1, keepdims=True)
        mean_pos = jnp.sum(yl * positions, axis=1, keepdims=True) / ysum
        var = jnp.sum(yl * (positions - mean_pos) ** 2,
                      axis=1, keepdims=True) / ysum
        spacing = jnp.sqrt(jnp.maximum(var, _EPS))

        s = jnp.zeros((1, Tc), f32)
        for it in range(Tc // _TILE_I):
            i0 = it * _TILE_I
            if _stage == 0:
                ui = u[i0:i0 + _TILE_I]
                gram = _dotg(ui, u, ((1,), (1,)))
                sqd = jnp.maximum(
                    sq_col[i0:i0 + _TILE_I] + sq_row - 2.0 * gram, 0.0)
                dist = jnp.sqrt(sqd + _EPS)
                dist_scr[i0:i0 + _TILE_I, :] = dist
            else:
                dist = dist_scr[i0:i0 + _TILE_I, :]
            s = s + _dotg(yl[:, i0:i0 + _TILE_I], dist, ((1,), (0,)))
        wdist = jnp.sum(s * yl, axis=1, keepdims=True)
        compactness = wdist / (ysum * ysum)

        c = jnp.concatenate([coverage, entropy, spacing, compactness], axis=1)
        mc = jnp.concatenate([m, c], axis=1)
        m = jnp.tanh(_dotg(mc, wm_ref[...], ((1,), (1,))) + bm_ref[...])
        prev = prev + yl
    out_ref[0] = yl


@jax.jit
def kernel(x, W_proj, b_proj, W_q, W_k, W_v, b_pos, log_temperature, m0,
           W_m, b_m):
    del W_v
    Bc, Tc, IN = x.shape
    DU = W_proj.shape[0]
    DM = m0.shape[0]
    bproj2 = b_proj.reshape(1, DU)
    bpos2 = b_pos[:Tc].reshape(1, Tc)
    lt2 = log_temperature.reshape(1, 1).astype(jnp.float32)
    m02 = m0.reshape(1, DM)
    bm2 = b_m.reshape(1, DM)
    out = pl.pallas_call(
        _router_kernel,
        grid=(Bc,),
        in_specs=[
            pl.BlockSpec((1, Tc, IN), lambda b: (b, 0, 0)),
            pl.BlockSpec(W_proj.shape, lambda b: (0, 0)),
            pl.BlockSpec((1, DU), lambda b: (0, 0)),
            pl.BlockSpec(W_q.shape, lambda b: (0, 0)),
            pl.BlockSpec(W_k.shape, lambda b: (0, 0)),
            pl.BlockSpec((1, Tc), lambda b: (0, 0)),
            pl.BlockSpec((1, 1), lambda b: (0, 0)),
            pl.BlockSpec((1, DM), lambda b: (0, 0)),
            pl.BlockSpec(W_m.shape, lambda b: (0, 0)),
            pl.BlockSpec((1, DM), lambda b: (0, 0)),
        ],
        out_specs=pl.BlockSpec((1, 1, Tc), lambda b: (b, 0, 0)),
        out_shape=jax.ShapeDtypeStruct((Bc, 1, Tc), jnp.float32),
        scratch_shapes=[pltpu.VMEM((Tc, Tc), jnp.float32)],
    )(x, W_proj, bproj2, W_q, W_k, bpos2, lt2, m02, W_m, bm2)
    return out.reshape(Bc, Tc)


# speedup vs baseline: 1.2104x; 1.2104x over previous
import math

import jax
import jax.numpy as jnp
from jax.experimental import pallas as pl
from jax.experimental.pallas import tpu as pltpu

_K_BUDGET = 128.0
_NUM_STAGES = 2
_R = 2
_GAMMA = 1.0
_EPS = 1e-6
_TILE_I = 512


def _dotg(a, b, contract):
    return jax.lax.dot_general(
        a, b, (contract, ((), ())), preferred_element_type=jnp.float32
    )


def _router_kernel(x_ref, wproj_ref, bproj_ref, wq_ref, wk_ref, bpos_ref,
                   lt_ref, m0_ref, wm_ref, bm_ref, out_ref, dist_scr):
    f32 = jnp.float32
    Tc = x_ref.shape[1]
    scale_a = math.sqrt(wq_ref.shape[0])

    xb = x_ref[0]
    u = _dotg(xb, wproj_ref[...], ((1,), (1,))) + bproj_ref[...]

    usq = u * u
    sq_col = jnp.sum(usq, axis=1, keepdims=True)
    ones_row = jnp.ones((1, u.shape[1]), f32)
    sq_row = _dotg(ones_row, usq, ((1,), (1,)))

    temp = jnp.clip(jnp.exp(lt_ref[0, 0]), 0.1, 10.0)
    lane_ids = jax.lax.broadcasted_iota(jnp.int32, (1, Tc), 1)
    positions = lane_ids.astype(f32)

    m = m0_ref[...]
    prev = jnp.zeros((1, Tc), f32)
    yl = prev
    for _stage in range(_NUM_STAGES):
        q = _dotg(m, wq_ref[...], ((1,), (1,)))
        k = _dotg(u, wk_ref[...], ((1,), (1,)))
        scores = _dotg(q, k, ((1,), (1,))) / scale_a
        scores = scores + bpos_ref[...] - _GAMMA * prev
        yl = jax.nn.sigmoid(scores / temp)
        budget = jnp.maximum(jnp.sum(yl), 1e-6)
        yl = yl * jnp.minimum(_K_BUDGET / budget, 1.0)
        for d in range(1, _R + 1):
            shift = jnp.concatenate([yl[:, d:], yl[:, :d]], axis=1)
            yl = yl * jnp.minimum(2.0 / (1.0 + yl + shift), 1.0)
        yl = jnp.where(lane_ids == 0, 0.0, yl)

        ssum = jnp.sum(yl, axis=1, keepdims=True)
        coverage = ssum / Tc
        ysum = jnp.maximum(ssum, _EPS)
        ynorm = yl / ysum
        entropy = -jnp.sum(ynorm * jnp.log(jnp.maximum(ynorm, _EPS)),
                           axis=1, keepdims=True)
        mean_pos = jnp.sum(yl * positions, axis=1, keepdims=True) / ysum
        var = jnp.sum(yl * (positions - mean_pos) ** 2,
                      axis=1, keepdims=True) / ysum
        spacing = jnp.sqrt(jnp.maximum(var, _EPS))

        s = jnp.zeros((1, Tc), f32)
        for it in range(Tc // _TILE_I):
            i0 = it * _TILE_I
            if _stage == 0:
                ui = u[i0:i0 + _TILE_I]
                gram = _dotg(ui, u, ((1,), (1,)))
                sqd = jnp.maximum(
                    sq_col[i0:i0 + _TILE_I] + sq_row - 2.0 * gram, 0.0)
                t = sqd + _EPS
                dist = t * jax.lax.rsqrt(t)
                dist_scr[i0:i0 + _TILE_I, :] = dist
            else:
                dist = dist_scr[i0:i0 + _TILE_I, :]
            s = s + _dotg(yl[:, i0:i0 + _TILE_I], dist, ((1,), (0,)))
        wdist = jnp.sum(s * yl, axis=1, keepdims=True)
        compactness = wdist / (ysum * ysum)

        c = jnp.concatenate([coverage, entropy, spacing, compactness], axis=1)
        mc = jnp.concatenate([m, c], axis=1)
        m = jnp.tanh(_dotg(mc, wm_ref[...], ((1,), (1,))) + bm_ref[...])
        prev = prev + yl
    out_ref[0] = yl


@jax.jit
def kernel(x, W_proj, b_proj, W_q, W_k, W_v, b_pos, log_temperature, m0,
           W_m, b_m):
    del W_v
    Bc, Tc, IN = x.shape
    DU = W_proj.shape[0]
    DM = m0.shape[0]
    bproj2 = b_proj.reshape(1, DU)
    bpos2 = b_pos[:Tc].reshape(1, Tc)
    lt2 = log_temperature.reshape(1, 1).astype(jnp.float32)
    m02 = m0.reshape(1, DM)
    bm2 = b_m.reshape(1, DM)
    out = pl.pallas_call(
        _router_kernel,
        grid=(Bc,),
        in_specs=[
            pl.BlockSpec((1, Tc, IN), lambda b: (b, 0, 0)),
            pl.BlockSpec(W_proj.shape, lambda b: (0, 0)),
            pl.BlockSpec((1, DU), lambda b: (0, 0)),
            pl.BlockSpec(W_q.shape, lambda b: (0, 0)),
            pl.BlockSpec(W_k.shape, lambda b: (0, 0)),
            pl.BlockSpec((1, Tc), lambda b: (0, 0)),
            pl.BlockSpec((1, 1), lambda b: (0, 0)),
            pl.BlockSpec((1, DM), lambda b: (0, 0)),
            pl.BlockSpec(W_m.shape, lambda b: (0, 0)),
            pl.BlockSpec((1, DM), lambda b: (0, 0)),
        ],
        out_specs=pl.BlockSpec((1, 1, Tc), lambda b: (b, 0, 0)),
        out_shape=jax.ShapeDtypeStruct((Bc, 1, Tc), jnp.float32),
        scratch_shapes=[pltpu.VMEM((Tc, Tc), jnp.float32)],
    )(x, W_proj, bproj2, W_q, W_k, bpos2, lt2, m02, W_m, bm2)
    return out.reshape(Bc, Tc)


# speedup vs baseline: 1.3560x; 1.1203x over previous
import math

import jax
import jax.numpy as jnp
from jax.experimental import pallas as pl
from jax.experimental.pallas import tpu as pltpu

_K_BUDGET = 128.0
_NUM_STAGES = 2
_R = 2
_GAMMA = 1.0
_EPS = 1e-6
_TILE_I = 512


def _dotg(a, b, contract):
    return jax.lax.dot_general(
        a, b, (contract, ((), ())), preferred_element_type=jnp.float32
    )


def _router_kernel(x_ref, wproj_ref, bproj_ref, wq_ref, wk_ref, bpos_ref,
                   lt_ref, m0_ref, wm_ref, bm_ref, out_ref, dist_scr):
    f32 = jnp.float32
    Tc = x_ref.shape[1]
    scale_a = math.sqrt(wq_ref.shape[0])

    xb = x_ref[0]
    u = _dotg(xb, wproj_ref[...], ((1,), (1,))) + bproj_ref[...]

    usq = u * u
    u2 = u + u
    sq_col = jnp.sum(usq, axis=1, keepdims=True)
    ones_row = jnp.ones((1, u.shape[1]), f32)
    sq_row_eps = _dotg(ones_row, usq, ((1,), (1,))) + _EPS
    row_id = jax.lax.broadcasted_iota(jnp.int32, (_TILE_I, _TILE_I), 0)
    col_id = jax.lax.broadcasted_iota(jnp.int32, (_TILE_I, _TILE_I), 1)
    upper_mask = row_id < col_id

    temp = jnp.clip(jnp.exp(lt_ref[0, 0]), 0.1, 10.0)
    lane_ids = jax.lax.broadcasted_iota(jnp.int32, (1, Tc), 1)
    positions = lane_ids.astype(f32)

    m = m0_ref[...]
    prev = jnp.zeros((1, Tc), f32)
    yl = prev
    for _stage in range(_NUM_STAGES):
        q = _dotg(m, wq_ref[...], ((1,), (1,)))
        k = _dotg(u, wk_ref[...], ((1,), (1,)))
        scores = _dotg(q, k, ((1,), (1,))) / scale_a
        scores = scores + bpos_ref[...] - _GAMMA * prev
        yl = jax.nn.sigmoid(scores / temp)
        budget = jnp.maximum(jnp.sum(yl), 1e-6)
        yl = yl * jnp.minimum(_K_BUDGET / budget, 1.0)
        for d in range(1, _R + 1):
            shift = jnp.concatenate([yl[:, d:], yl[:, :d]], axis=1)
            yl = yl * jnp.minimum(2.0 / (1.0 + yl + shift), 1.0)
        yl = jnp.where(lane_ids == 0, 0.0, yl)

        ssum = jnp.sum(yl, axis=1, keepdims=True)
        coverage = ssum / Tc
        ysum = jnp.maximum(ssum, _EPS)
        ynorm = yl / ysum
        entropy = -jnp.sum(ynorm * jnp.log(jnp.maximum(ynorm, _EPS)),
                           axis=1, keepdims=True)
        mean_pos = jnp.sum(yl * positions, axis=1, keepdims=True) / ysum
        var = jnp.sum(yl * (positions - mean_pos) ** 2,
                      axis=1, keepdims=True) / ysum
        spacing = jnp.sqrt(jnp.maximum(var, _EPS))

        nt = Tc // _TILE_I
        acc = jnp.zeros((1, 1), f32)
        tidx = 0
        for ti in range(nt):
            i0 = ti * _TILE_I
            for tj in range(ti, nt):
                j0 = tj * _TILE_I
                if _stage == 0:
                    gram2 = _dotg(u2[i0:i0 + _TILE_I],
                                  u[j0:j0 + _TILE_I], ((1,), (1,)))
                    t = jnp.maximum(
                        sq_col[i0:i0 + _TILE_I]
                        + sq_row_eps[:, j0:j0 + _TILE_I] - gram2, _EPS)
                    dist = t * jax.lax.rsqrt(t)
                    if ti == tj:
                        dist = jnp.where(upper_mask, dist, 0.0)
                    dist_scr[tidx * _TILE_I:(tidx + 1) * _TILE_I, :] = dist
                else:
                    dist = dist_scr[tidx * _TILE_I:(tidx + 1) * _TILE_I, :]
                v = _dotg(yl[:, i0:i0 + _TILE_I], dist, ((1,), (0,)))
                acc = acc + jnp.sum(v * yl[:, j0:j0 + _TILE_I],
                                    axis=1, keepdims=True)
                tidx += 1
        wdist = (2.0 * acc
                 + math.sqrt(_EPS) * jnp.sum(yl * yl, axis=1, keepdims=True))
        compactness = wdist / (ysum * ysum)

        c = jnp.concatenate([coverage, entropy, spacing, compactness], axis=1)
        mc = jnp.concatenate([m, c], axis=1)
        m = jnp.tanh(_dotg(mc, wm_ref[...], ((1,), (1,))) + bm_ref[...])
        prev = prev + yl
    out_ref[0] = yl


@jax.jit
def kernel(x, W_proj, b_proj, W_q, W_k, W_v, b_pos, log_temperature, m0,
           W_m, b_m):
    del W_v
    Bc, Tc, IN = x.shape
    DU = W_proj.shape[0]
    DM = m0.shape[0]
    bproj2 = b_proj.reshape(1, DU)
    bpos2 = b_pos[:Tc].reshape(1, Tc)
    lt2 = log_temperature.reshape(1, 1).astype(jnp.float32)
    m02 = m0.reshape(1, DM)
    bm2 = b_m.reshape(1, DM)
    out = pl.pallas_call(
        _router_kernel,
        grid=(Bc,),
        in_specs=[
            pl.BlockSpec((1, Tc, IN), lambda b: (b, 0, 0)),
            pl.BlockSpec(W_proj.shape, lambda b: (0, 0)),
            pl.BlockSpec((1, DU), lambda b: (0, 0)),
            pl.BlockSpec(W_q.shape, lambda b: (0, 0)),
            pl.BlockSpec(W_k.shape, lambda b: (0, 0)),
            pl.BlockSpec((1, Tc), lambda b: (0, 0)),
            pl.BlockSpec((1, 1), lambda b: (0, 0)),
            pl.BlockSpec((1, DM), lambda b: (0, 0)),
            pl.BlockSpec(W_m.shape, lambda b: (0, 0)),
            pl.BlockSpec((1, DM), lambda b: (0, 0)),
        ],
        out_specs=pl.BlockSpec((1, 1, Tc), lambda b: (b, 0, 0)),
        out_shape=jax.ShapeDtypeStruct((Bc, 1, Tc), jnp.float32),
        scratch_shapes=[pltpu.VMEM(
            (_TILE_I * ((Tc // _TILE_I) * (Tc // _TILE_I + 1) // 2), _TILE_I),
            jnp.float32)],
    )(x, W_proj, bproj2, W_q, W_k, bpos2, lt2, m02, W_m, bm2)
    return out.reshape(Bc, Tc)
